# baseline (speedup 1.0000x reference)
import functools
import math

import jax
import jax.numpy as jnp
from jax import lax
from jax.experimental import pallas as pl
from jax.experimental.pallas import tpu as pltpu
from jax.experimental.pallas import tpu_sc as plsc

B, N, IN_DIM, E, H, K, FF, OUT = 2, 2048, 3, 256, 8, 32, 512, 256
DH = E // H
BIG = 3.0e38

_R1 = 2048


def _pack16(a):
    bits = lax.bitcast_convert_type(a, jnp.uint32) + jnp.uint32(0x8000)
    lo = bits[:, :128] >> 16
    hi = bits[:, 128:] & jnp.uint32(0xFFFF0000)
    return lax.bitcast_convert_type(lo | hi, jnp.int32)


def _unpack16_halves(p):
    lo = lax.bitcast_convert_type(p << 16, jnp.float32)
    hi = lax.bitcast_convert_type(p & jnp.int32(-65536), jnp.float32)
    return lo, hi


def _prep_body(x_ref, xv_ref, wemb_ref, wq_ref, wk_ref, wv_ref, bq_ref,
               bk_ref, bv_ref, h0_ref, q_ref, kv_ref, xvt_ref):
    eye3 = (lax.broadcasted_iota(jnp.int32, (IN_DIM, IN_DIM), 0) ==
            lax.broadcasted_iota(jnp.int32, (IN_DIM, IN_DIM), 1)).astype(jnp.float32)
    xvt_ref[0] = lax.dot_general(eye3, xv_ref[...], (((1,), (1,)), ((), ())),
                                 precision=lax.Precision.HIGHEST,
                                 preferred_element_type=jnp.float32)
    x = x_ref[...]
    h0 = lax.dot_general(x, wemb_ref[...], (((1,), (1,)), ((), ())),
                         preferred_element_type=jnp.float32)
    h0_ref[...] = h0
    q_ref[...] = lax.dot_general(h0, wq_ref[...], (((1,), (1,)), ((), ())),
                                 preferred_element_type=jnp.float32) + bq_ref[...]
    kp = lax.dot_general(h0, wk_ref[...], (((1,), (1,)), ((), ())),
                         preferred_element_type=jnp.float32) + bk_ref[...]
    vp = lax.dot_general(h0, wv_ref[...], (((1,), (1,)), ((), ())),
                         preferred_element_type=jnp.float32) + bv_ref[...]
    kv_ref[:, 0:128] = _pack16(kp)
    kv_ref[:, 128:256] = _pack16(vp)


def _prep(x2, xv2, wemb, wq, wk, wv, bq, bk, bv):
    nrow = x2.shape[0]
    grid = (nrow // _R1,)
    row_spec = pl.BlockSpec((_R1, E), lambda i: (i, 0))
    full = lambda s: pl.BlockSpec(s, lambda i: (0,) * len(s))
    return pl.pallas_call(
        _prep_body,
        grid=grid,
        in_specs=[
            pl.BlockSpec((_R1, IN_DIM), lambda i: (i, 0)),
            pl.BlockSpec((_R1, IN_DIM), lambda i: (i, 0)),
            full((E, IN_DIM)), full((E, E)), full((E, E)), full((E, E)),
            full((1, E)), full((1, E)), full((1, E)),
        ],
        out_specs=[row_spec, row_spec,
                   pl.BlockSpec((_R1, E), lambda i: (i, 0)),
                   pl.BlockSpec((1, IN_DIM, N), lambda i: (i, 0, 0))],
        out_shape=[jax.ShapeDtypeStruct((nrow, E), jnp.float32)] * 2 +
                  [jax.ShapeDtypeStruct((nrow, E), jnp.int32),
                   jax.ShapeDtypeStruct((nrow // N, IN_DIM, N), jnp.float32)],
    )(x2, xv2, wemb, wq, wk, wv, bq, bk, bv)


_R2 = 256


def _knn_batch(xv2, xvt, b, row0, nrow):
    def body(xvr_ref, xvct_ref, idx_ref):
        xr = xvr_ref[...]
        xct = xvct_ref[0]
        d = None
        for c in range(IN_DIM):
            diff = xr[:, c:c + 1] - xct[c:c + 1, :]
            sq = diff * diff
            d = sq if d is None else d + sq
        iota = lax.broadcasted_iota(jnp.int32, (_R2, N), 1)
        key = (lax.bitcast_convert_type(d, jnp.int32) & ~jnp.int32(0x7FF)) | iota
        cols = []
        for _ in range(K):
            m = jnp.min(key, axis=1, keepdims=True)
            cols.append(m & 0x7FF)
            key = jnp.where(key == m, jnp.int32(0x7FFFFFFF), key)
        idx_ref[...] = jnp.concatenate(cols, axis=1) + b * N

    blk0 = row0 // _R2
    return pl.pallas_call(
        body,
        grid=(nrow // _R2,),
        in_specs=[
            pl.BlockSpec((_R2, IN_DIM), lambda i: (blk0 + i, 0)),
            pl.BlockSpec((1, IN_DIM, N), lambda i: (b, 0, 0)),
        ],
        out_specs=pl.BlockSpec((_R2, K), lambda i: (i, 0)),
        out_shape=jax.ShapeDtypeStruct((nrow, K), jnp.int32),
    )(xv2, xvt)


_NW = 32
_CH = 128


def _sc_gather(kv2, idxg):
    nidx = idxg.shape[0] * 128
    per_w = nidx // _NW
    nchunk = per_w // _CH
    mesh = plsc.VectorSubcoreMesh(core_axis_name="c", subcore_axis_name="s")

    @functools.partial(
        pl.kernel,
        mesh=mesh,
        out_type=jax.ShapeDtypeStruct((nidx, E), jnp.int32),
        scratch_types=[
            pltpu.VMEM((nchunk, _CH), jnp.int32),
            pltpu.VMEM((_CH, E), jnp.int32),
            pltpu.VMEM((_CH, E), jnp.int32),
            pltpu.SemaphoreType.DMA,
            pltpu.SemaphoreType.DMA,
            pltpu.SemaphoreType.DMA,
            pltpu.SemaphoreType.DMA,
        ],
    )
    def k(kv_hbm, idx_hbm, gkv_hbm, idx_all, buf0, buf1, g0, g1, s0, s1):
        wid = lax.axis_index("s") * 2 + lax.axis_index("c")
        base = wid * per_w
        pltpu.sync_copy(idx_hbm.at[pl.ds(wid * nchunk, nchunk)], idx_all)
        bufs = (buf0, buf1)
        gsems = (g0, g1)
        ssems = (s0, s1)

        def gath(cn, buf, gsem):
            pltpu.async_copy(kv_hbm.at[idx_all.at[cn]], buf, gsem)

        gath(0, buf0, g0)
        gath(1, buf1, g1)

        def body(i, carry):
            for sl in range(2):
                cn = 2 * i + sl
                buf, gsem, ssem = bufs[sl], gsems[sl], ssems[sl]
                pltpu.make_async_copy(kv_hbm.at[pl.ds(0, _CH)],
                                      buf, gsem).wait()
                cp = pltpu.async_copy(
                    buf, gkv_hbm.at[pl.ds(base + cn * _CH, _CH)], ssem)
                cp.wait()

                @pl.when(cn + 2 < nchunk)
                def _():
                    gath(cn + 2, buf, gsem)

            return carry

        lax.fori_loop(0, nchunk // 2, body, 0)

    return k(kv2, idxg)


_R4 = 256


def _ln_in(xv, g, b):
    m = jnp.mean(xv, axis=1, keepdims=True)
    c = xv - m
    v = jnp.mean(c * c, axis=1, keepdims=True)
    return c * lax.rsqrt(v + 1e-5) * g + b


def _attn_body(q_ref, h0_ref, gkv_ref, ow_ref, ob_ref, ln1g_ref,
               ln1b_ref, fw1_ref, fb1_ref, fw2_ref, fb2_ref, ln2g_ref,
               ln2b_ref, wout_ref, out_ref):
    q = q_ref[...]
    gkv = gkv_ref[...]
    gk_lo, gk_hi = _unpack16_halves(gkv[:, 0:128])
    gv_lo, gv_hi = _unpack16_halves(gkv[:, 128:256])
    seg_lo = (lax.broadcasted_iota(jnp.int32, (128, H), 0) // DH ==
              lax.broadcasted_iota(jnp.int32, (128, H), 1)).astype(jnp.float32)
    seg_hi = ((lax.broadcasted_iota(jnp.int32, (128, H), 0) // DH + 4) ==
              lax.broadcasted_iota(jnp.int32, (128, H), 1)).astype(jnp.float32)
    qb_lo = jnp.broadcast_to(q[:, None, 0:128],
                             (_R4, K, 128)).reshape(_R4 * K, 128)
    qb_hi = jnp.broadcast_to(q[:, None, 128:256],
                             (_R4, K, 128)).reshape(_R4 * K, 128)
    l = (lax.dot_general(gk_lo * qb_lo, seg_lo, (((1,), (0,)), ((), ())),
                         preferred_element_type=jnp.float32) +
         lax.dot_general(gk_hi * qb_hi, seg_hi, (((1,), (0,)), ((), ())),
                         preferred_element_type=jnp.float32)) * (1.0 / math.sqrt(DH))
    l3 = l.reshape(_R4, K, H)
    m = jnp.max(l3, axis=1, keepdims=True)
    e = jnp.exp(l3 - m)
    s = jnp.sum(e, axis=1, keepdims=True)
    w3 = e / s
    w2 = w3.reshape(_R4 * K, H)
    wex_lo = lax.dot_general(w2, seg_lo, (((1,), (1,)), ((), ())),
                             preferred_element_type=jnp.float32)
    wex_hi = lax.dot_general(w2, seg_hi, (((1,), (1,)), ((), ())),
                             preferred_element_type=jnp.float32)
    o_lo = jnp.sum((wex_lo * gv_lo).reshape(_R4, K, 128), axis=1)
    o_hi = jnp.sum((wex_hi * gv_hi).reshape(_R4, K, 128), axis=1)
    o = jnp.concatenate([o_lo, o_hi], axis=1)
    mo = lax.dot_general(o, ow_ref[...], (((1,), (1,)), ((), ())),
                         preferred_element_type=jnp.float32) + ob_ref[...]
    h1 = _ln_in(mo + h0_ref[...], ln1g_ref[...], ln1b_ref[...])
    ffh = jax.nn.gelu(lax.dot_general(h1, fw1_ref[...], (((1,), (1,)), ((), ())),
                                      preferred_element_type=jnp.float32)
                      + fb1_ref[...])
    ffo = lax.dot_general(ffh, fw2_ref[...], (((1,), (1,)), ((), ())),
                          preferred_element_type=jnp.float32) + fb2_ref[...]
    h2 = _ln_in(ffo + h1, ln2g_ref[...], ln2b_ref[...])
    out_ref[...] = lax.dot_general(h2, wout_ref[...], (((1,), (1,)), ((), ())),
                                   preferred_element_type=jnp.float32)


def _attn(q2, h02, gkv, blk0, ow, ob, ln1g, ln1b, fw1, fb1, fw2, fb2, ln2g,
          ln2b, wout):
    nrow = gkv.shape[0] // K
    row = pl.BlockSpec((_R4, E), lambda i: (blk0 + i, 0))
    grow = pl.BlockSpec((_R4 * K, E), lambda i: (i, 0))
    full = lambda s: pl.BlockSpec(s, lambda i: (0,) * len(s))
    return pl.pallas_call(
        _attn_body,
        grid=(nrow // _R4,),
        in_specs=[
            row, row, grow,
            full((E, E)), full((1, E)), full((1, E)), full((1, E)),
            full((FF, E)), full((1, FF)), full((E, FF)), full((1, E)),
            full((1, E)), full((1, E)), full((OUT, E)),
        ],
        out_specs=pl.BlockSpec((_R4, OUT), lambda i: (i, 0)),
        out_shape=jax.ShapeDtypeStruct((nrow, OUT), jnp.float32),
    )(q2, h02, gkv, ow, ob, ln1g, ln1b, fw1, fb1, fw2, fb2, ln2g, ln2b,
      wout)


def kernel(x, x_v, W_emb, in_proj_w, in_proj_b, out_proj_w, out_proj_b,
           ln1_g, ln1_b, ff_w1, ff_b1, ff_w2, ff_b2, ln2_g, ln2_b, W_out):
    x2 = x.reshape(B * N, IN_DIM)
    wq, wk, wv = in_proj_w[0:E], in_proj_w[E:2 * E], in_proj_w[2 * E:3 * E]
    bq = in_proj_b[0:E].reshape(1, E)
    bk = in_proj_b[E:2 * E].reshape(1, E)
    bv = in_proj_b[2 * E:3 * E].reshape(1, E)
    xv2 = x_v.reshape(B * N, IN_DIM)
    h02, q2, kvp, xvt = _prep(x2, xv2, W_emb, wq, wk, wv, bq, bk, bv)
    nslc = 1
    rp = N // nslc
    idxs = []
    for b in range(B):
        for h in range(nslc):
            idxs.append(_knn_batch(xv2, xvt, b, b * N + h * rp, rp))
    gkvs = [_sc_gather(kvp, ix.reshape(-1, 128)) for ix in idxs]
    outs = [
        _attn(q2, h02, gkvs[s], s * (rp // _R4), out_proj_w,
              out_proj_b.reshape(1, E), ln1_g.reshape(1, E),
              ln1_b.reshape(1, E), ff_w1, ff_b1.reshape(1, FF), ff_w2,
              ff_b2.reshape(1, E), ln2_g.reshape(1, E), ln2_b.reshape(1, E),
              W_out)
        for s in range(B * nslc)
    ]
    out2 = jnp.concatenate(outs, axis=0)
    return out2.reshape(B, N, OUT), x_v


# speedup vs baseline: 1.1077x; 1.1077x over previous
import functools
import math

import jax
import jax.numpy as jnp
from jax import lax
from jax.experimental import pallas as pl
from jax.experimental.pallas import tpu as pltpu
from jax.experimental.pallas import tpu_sc as plsc

B, N, IN_DIM, E, H, K, FF, OUT = 2, 2048, 3, 256, 8, 32, 512, 256
DH = E // H
BIG = 3.0e38

_R1 = 2048


def _pack16(a):
    bits = lax.bitcast_convert_type(a, jnp.uint32) + jnp.uint32(0x8000)
    lo = bits[:, :128] >> 16
    hi = bits[:, 128:] & jnp.uint32(0xFFFF0000)
    return lax.bitcast_convert_type(lo | hi, jnp.int32)


def _unpack16_halves(p):
    lo = lax.bitcast_convert_type(p << 16, jnp.float32)
    hi = lax.bitcast_convert_type(p & jnp.int32(-65536), jnp.float32)
    return lo, hi


def _prep_body(x_ref, xv_ref, wemb_ref, wq_ref, wk_ref, wv_ref, bq_ref,
               bk_ref, bv_ref, h0_ref, q_ref, kv_ref, xvt_ref):
    eye3 = (lax.broadcasted_iota(jnp.int32, (IN_DIM, IN_DIM), 0) ==
            lax.broadcasted_iota(jnp.int32, (IN_DIM, IN_DIM), 1)).astype(jnp.float32)
    xvt_ref[0] = lax.dot_general(eye3, xv_ref[...], (((1,), (1,)), ((), ())),
                                 precision=lax.Precision.HIGHEST,
                                 preferred_element_type=jnp.float32)
    x = x_ref[...]
    h0 = lax.dot_general(x, wemb_ref[...], (((1,), (1,)), ((), ())),
                         preferred_element_type=jnp.float32)
    h0_ref[...] = h0
    q_ref[...] = lax.dot_general(h0, wq_ref[...], (((1,), (1,)), ((), ())),
                                 preferred_element_type=jnp.float32) + bq_ref[...]
    kp = lax.dot_general(h0, wk_ref[...], (((1,), (1,)), ((), ())),
                         preferred_element_type=jnp.float32) + bk_ref[...]
    vp = lax.dot_general(h0, wv_ref[...], (((1,), (1,)), ((), ())),
                         preferred_element_type=jnp.float32) + bv_ref[...]
    kv_ref[:, 0:128] = _pack16(kp)
    kv_ref[:, 128:256] = _pack16(vp)


def _prep(x2, xv2, wemb, wq, wk, wv, bq, bk, bv):
    nrow = x2.shape[0]
    grid = (nrow // _R1,)
    row_spec = pl.BlockSpec((_R1, E), lambda i: (i, 0))
    full = lambda s: pl.BlockSpec(s, lambda i: (0,) * len(s))
    return pl.pallas_call(
        _prep_body,
        grid=grid,
        in_specs=[
            pl.BlockSpec((_R1, IN_DIM), lambda i: (i, 0)),
            pl.BlockSpec((_R1, IN_DIM), lambda i: (i, 0)),
            full((E, IN_DIM)), full((E, E)), full((E, E)), full((E, E)),
            full((1, E)), full((1, E)), full((1, E)),
        ],
        out_specs=[row_spec, row_spec,
                   pl.BlockSpec((_R1, E), lambda i: (i, 0)),
                   pl.BlockSpec((1, IN_DIM, N), lambda i: (i, 0, 0))],
        out_shape=[jax.ShapeDtypeStruct((nrow, E), jnp.float32)] * 2 +
                  [jax.ShapeDtypeStruct((nrow, E), jnp.int32),
                   jax.ShapeDtypeStruct((nrow // N, IN_DIM, N), jnp.float32)],
    )(x2, xv2, wemb, wq, wk, wv, bq, bk, bv)


_R2 = 256


def _knn_batch(xv2, xvt, b, row0, nrow):
    row0b = row0 - b * N
    maxi = 0x7FFFFFFF

    def body(xvr_ref, xvct_ref, idx_ref):
        xr = xvr_ref[...]
        xct = xvct_ref[0]
        d = None
        for c in range(IN_DIM):
            diff = xr[:, c:c + 1] - xct[c:c + 1, :]
            sq = diff * diff
            d = sq if d is None else d + sq
        iota = lax.broadcasted_iota(jnp.int32, (_R2, N), 1)
        key = (lax.bitcast_convert_type(d, jnp.int32) & ~jnp.int32(0x7FF)) | iota
        riota = (lax.broadcasted_iota(jnp.int32, (_R2, 1), 0) + row0b
                 + pl.program_id(0) * _R2)
        key = jnp.where(iota == riota, maxi, key)
        s0 = jnp.minimum(key[:, :N // 2], key[:, N // 2:])
        s1 = jnp.maximum(key[:, :N // 2], key[:, N // 2:])
        cols = [riota]
        for _ in range(K - 1):
            m = jnp.min(s0, axis=1, keepdims=True)
            cols.append(m & 0x7FF)
            mask = s0 == m
            s0 = jnp.where(mask, s1, s0)
            s1 = jnp.where(mask, maxi, s1)
        idx_ref[...] = jnp.concatenate(cols, axis=1) + b * N

    blk0 = row0 // _R2
    return pl.pallas_call(
        body,
        grid=(nrow // _R2,),
        in_specs=[
            pl.BlockSpec((_R2, IN_DIM), lambda i: (blk0 + i, 0)),
            pl.BlockSpec((1, IN_DIM, N), lambda i: (b, 0, 0)),
        ],
        out_specs=pl.BlockSpec((_R2, K), lambda i: (i, 0)),
        out_shape=jax.ShapeDtypeStruct((nrow, K), jnp.int32),
    )(xv2, xvt)


_NW = 32
_CH = 128


def _sc_gather(kv2, idxg):
    nidx = idxg.shape[0] * 128
    per_w = nidx // _NW
    nchunk = per_w // _CH
    mesh = plsc.VectorSubcoreMesh(core_axis_name="c", subcore_axis_name="s")

    @functools.partial(
        pl.kernel,
        mesh=mesh,
        out_type=jax.ShapeDtypeStruct((nidx, E), jnp.int32),
        scratch_types=[
            pltpu.VMEM((nchunk, _CH), jnp.int32),
            pltpu.VMEM((_CH, E), jnp.int32),
            pltpu.VMEM((_CH, E), jnp.int32),
            pltpu.SemaphoreType.DMA,
            pltpu.SemaphoreType.DMA,
            pltpu.SemaphoreType.DMA,
            pltpu.SemaphoreType.DMA,
        ],
    )
    def k(kv_hbm, idx_hbm, gkv_hbm, idx_all, buf0, buf1, g0, g1, s0, s1):
        wid = lax.axis_index("s") * 2 + lax.axis_index("c")
        base = wid * per_w
        pltpu.sync_copy(idx_hbm.at[pl.ds(wid * nchunk, nchunk)], idx_all)
        bufs = (buf0, buf1)
        gsems = (g0, g1)
        ssems = (s0, s1)

        def gath(cn, buf, gsem):
            pltpu.async_copy(kv_hbm.at[idx_all.at[cn]], buf, gsem)

        gath(0, buf0, g0)
        gath(1, buf1, g1)

        def body(i, carry):
            for sl in range(2):
                cn = 2 * i + sl
                buf, gsem, ssem = bufs[sl], gsems[sl], ssems[sl]
                pltpu.make_async_copy(kv_hbm.at[pl.ds(0, _CH)],
                                      buf, gsem).wait()
                cp = pltpu.async_copy(
                    buf, gkv_hbm.at[pl.ds(base + cn * _CH, _CH)], ssem)
                cp.wait()

                @pl.when(cn + 2 < nchunk)
                def _():
                    gath(cn + 2, buf, gsem)

            return carry

        lax.fori_loop(0, nchunk // 2, body, 0)

    return k(kv2, idxg)


_R4 = 256


def _ln_in(xv, g, b):
    m = jnp.mean(xv, axis=1, keepdims=True)
    c = xv - m
    v = jnp.mean(c * c, axis=1, keepdims=True)
    return c * lax.rsqrt(v + 1e-5) * g + b


def _attn_body(q_ref, h0_ref, gkv_ref, ow_ref, ob_ref, ln1g_ref,
               ln1b_ref, fw1_ref, fb1_ref, fw2_ref, fb2_ref, ln2g_ref,
               ln2b_ref, wout_ref, out_ref):
    q = q_ref[...]
    gkv = gkv_ref[...]
    gk_lo, gk_hi = _unpack16_halves(gkv[:, 0:128])
    gv_lo, gv_hi = _unpack16_halves(gkv[:, 128:256])
    seg_lo = (lax.broadcasted_iota(jnp.int32, (128, H), 0) // DH ==
              lax.broadcasted_iota(jnp.int32, (128, H), 1)).astype(jnp.float32)
    seg_hi = ((lax.broadcasted_iota(jnp.int32, (128, H), 0) // DH + 4) ==
              lax.broadcasted_iota(jnp.int32, (128, H), 1)).astype(jnp.float32)
    qb_lo = jnp.broadcast_to(q[:, None, 0:128],
                             (_R4, K, 128)).reshape(_R4 * K, 128)
    qb_hi = jnp.broadcast_to(q[:, None, 128:256],
                             (_R4, K, 128)).reshape(_R4 * K, 128)
    l = (lax.dot_general(gk_lo * qb_lo, seg_lo, (((1,), (0,)), ((), ())),
                         preferred_element_type=jnp.float32) +
         lax.dot_general(gk_hi * qb_hi, seg_hi, (((1,), (0,)), ((), ())),
                         preferred_element_type=jnp.float32)) * (1.0 / math.sqrt(DH))
    l3 = l.reshape(_R4, K, H)
    m = jnp.max(l3, axis=1, keepdims=True)
    e = jnp.exp(l3 - m)
    s = jnp.sum(e, axis=1, keepdims=True)
    w3 = e / s
    w2 = w3.reshape(_R4 * K, H)
    wex_lo = lax.dot_general(w2, seg_lo, (((1,), (1,)), ((), ())),
                             preferred_element_type=jnp.float32)
    wex_hi = lax.dot_general(w2, seg_hi, (((1,), (1,)), ((), ())),
                             preferred_element_type=jnp.float32)
    o_lo = jnp.sum((wex_lo * gv_lo).reshape(_R4, K, 128), axis=1)
    o_hi = jnp.sum((wex_hi * gv_hi).reshape(_R4, K, 128), axis=1)
    o = jnp.concatenate([o_lo, o_hi], axis=1)
    mo = lax.dot_general(o, ow_ref[...], (((1,), (1,)), ((), ())),
                         preferred_element_type=jnp.float32) + ob_ref[...]
    h1 = _ln_in(mo + h0_ref[...], ln1g_ref[...], ln1b_ref[...])
    ffh = jax.nn.gelu(lax.dot_general(h1, fw1_ref[...], (((1,), (1,)), ((), ())),
                                      preferred_element_type=jnp.float32)
                      + fb1_ref[...])
    ffo = lax.dot_general(ffh, fw2_ref[...], (((1,), (1,)), ((), ())),
                          preferred_element_type=jnp.float32) + fb2_ref[...]
    h2 = _ln_in(ffo + h1, ln2g_ref[...], ln2b_ref[...])
    out_ref[...] = lax.dot_general(h2, wout_ref[...], (((1,), (1,)), ((), ())),
                                   preferred_element_type=jnp.float32)


def _attn(q2, h02, gkv, blk0, ow, ob, ln1g, ln1b, fw1, fb1, fw2, fb2, ln2g,
          ln2b, wout):
    nrow = gkv.shape[0] // K
    row = pl.BlockSpec((_R4, E), lambda i: (blk0 + i, 0))
    grow = pl.BlockSpec((_R4 * K, E), lambda i: (i, 0))
    full = lambda s: pl.BlockSpec(s, lambda i: (0,) * len(s))
    return pl.pallas_call(
        _attn_body,
        grid=(nrow // _R4,),
        in_specs=[
            row, row, grow,
            full((E, E)), full((1, E)), full((1, E)), full((1, E)),
            full((FF, E)), full((1, FF)), full((E, FF)), full((1, E)),
            full((1, E)), full((1, E)), full((OUT, E)),
        ],
        out_specs=pl.BlockSpec((_R4, OUT), lambda i: (i, 0)),
        out_shape=jax.ShapeDtypeStruct((nrow, OUT), jnp.float32),
    )(q2, h02, gkv, ow, ob, ln1g, ln1b, fw1, fb1, fw2, fb2, ln2g, ln2b,
      wout)


def kernel(x, x_v, W_emb, in_proj_w, in_proj_b, out_proj_w, out_proj_b,
           ln1_g, ln1_b, ff_w1, ff_b1, ff_w2, ff_b2, ln2_g, ln2_b, W_out):
    x2 = x.reshape(B * N, IN_DIM)
    wq, wk, wv = in_proj_w[0:E], in_proj_w[E:2 * E], in_proj_w[2 * E:3 * E]
    bq = in_proj_b[0:E].reshape(1, E)
    bk = in_proj_b[E:2 * E].reshape(1, E)
    bv = in_proj_b[2 * E:3 * E].reshape(1, E)
    xv2 = x_v.reshape(B * N, IN_DIM)
    h02, q2, kvp, xvt = _prep(x2, xv2, W_emb, wq, wk, wv, bq, bk, bv)
    nslc = 1
    rp = N // nslc
    idxs = []
    for b in range(B):
        for h in range(nslc):
            idxs.append(_knn_batch(xv2, xvt, b, b * N + h * rp, rp))
    gkvs = [_sc_gather(kvp, ix.reshape(-1, 128)) for ix in idxs]
    outs = [
        _attn(q2, h02, gkvs[s], s * (rp // _R4), out_proj_w,
              out_proj_b.reshape(1, E), ln1_g.reshape(1, E),
              ln1_b.reshape(1, E), ff_w1, ff_b1.reshape(1, FF), ff_w2,
              ff_b2.reshape(1, E), ln2_g.reshape(1, E), ln2_b.reshape(1, E),
              W_out)
        for s in range(B * nslc)
    ]
    out2 = jnp.concatenate(outs, axis=0)
    return out2.reshape(B, N, OUT), x_v


# speedup vs baseline: 1.1282x; 1.0185x over previous
import functools
import math

import jax
import jax.numpy as jnp
from jax import lax
from jax.experimental import pallas as pl
from jax.experimental.pallas import tpu as pltpu
from jax.experimental.pallas import tpu_sc as plsc

B, N, IN_DIM, E, H, K, FF, OUT = 2, 2048, 3, 256, 8, 32, 512, 256
DH = E // H
BIG = 3.0e38

_R1 = 2048


def _pack16(a):
    bits = lax.bitcast_convert_type(a, jnp.uint32) + jnp.uint32(0x8000)
    lo = bits[:, :128] >> 16
    hi = bits[:, 128:] & jnp.uint32(0xFFFF0000)
    return lax.bitcast_convert_type(lo | hi, jnp.int32)


def _unpack16_halves(p):
    lo = lax.bitcast_convert_type(p << 16, jnp.float32)
    hi = lax.bitcast_convert_type(p & jnp.int32(-65536), jnp.float32)
    return lo, hi


def _prep_body(x_ref, xv_ref, wemb_ref, wq_ref, wk_ref, wv_ref, bq_ref,
               bk_ref, bv_ref, h0_ref, q_ref, kv_ref, xvt_ref):
    eye3 = (lax.broadcasted_iota(jnp.int32, (IN_DIM, IN_DIM), 0) ==
            lax.broadcasted_iota(jnp.int32, (IN_DIM, IN_DIM), 1)).astype(jnp.float32)
    xvt_ref[0] = lax.dot_general(eye3, xv_ref[...], (((1,), (1,)), ((), ())),
                                 precision=lax.Precision.HIGHEST,
                                 preferred_element_type=jnp.float32)
    x = x_ref[...]
    h0 = lax.dot_general(x, wemb_ref[...], (((1,), (1,)), ((), ())),
                         preferred_element_type=jnp.float32)
    h0_ref[...] = h0
    q_ref[...] = lax.dot_general(h0, wq_ref[...], (((1,), (1,)), ((), ())),
                                 preferred_element_type=jnp.float32) + bq_ref[...]
    kp = lax.dot_general(h0, wk_ref[...], (((1,), (1,)), ((), ())),
                         preferred_element_type=jnp.float32) + bk_ref[...]
    vp = lax.dot_general(h0, wv_ref[...], (((1,), (1,)), ((), ())),
                         preferred_element_type=jnp.float32) + bv_ref[...]
    kv_ref[:, 0:128] = _pack16(kp)
    kv_ref[:, 128:256] = _pack16(vp)


def _prep(x2, xv2, wemb, wq, wk, wv, bq, bk, bv):
    nrow = x2.shape[0]
    grid = (nrow // _R1,)
    row_spec = pl.BlockSpec((_R1, E), lambda i: (i, 0))
    full = lambda s: pl.BlockSpec(s, lambda i: (0,) * len(s))
    return pl.pallas_call(
        _prep_body,
        grid=grid,
        in_specs=[
            pl.BlockSpec((_R1, IN_DIM), lambda i: (i, 0)),
            pl.BlockSpec((_R1, IN_DIM), lambda i: (i, 0)),
            full((E, IN_DIM)), full((E, E)), full((E, E)), full((E, E)),
            full((1, E)), full((1, E)), full((1, E)),
        ],
        out_specs=[row_spec, row_spec,
                   pl.BlockSpec((_R1, E), lambda i: (i, 0)),
                   pl.BlockSpec((1, IN_DIM, N), lambda i: (i, 0, 0))],
        out_shape=[jax.ShapeDtypeStruct((nrow, E), jnp.float32)] * 2 +
                  [jax.ShapeDtypeStruct((nrow, E), jnp.int32),
                   jax.ShapeDtypeStruct((nrow // N, IN_DIM, N), jnp.float32)],
    )(x2, xv2, wemb, wq, wk, wv, bq, bk, bv)


_R2 = 256


def _knn_batch(xv2, xvt, b, row0, nrow):
    row0b = row0 - b * N
    maxi = 0x7FFFFFFF

    def body(xvr_ref, xvct_ref, idx_ref):
        xr = xvr_ref[...]
        xct = xvct_ref[0]
        d = None
        for c in range(IN_DIM):
            diff = xr[:, c:c + 1] - xct[c:c + 1, :]
            sq = diff * diff
            d = sq if d is None else d + sq
        iota = lax.broadcasted_iota(jnp.int32, (_R2, N), 1)
        key = (lax.bitcast_convert_type(d, jnp.int32) & ~jnp.int32(0x7FF)) | iota
        riota = (lax.broadcasted_iota(jnp.int32, (_R2, 1), 0) + row0b
                 + pl.program_id(0) * _R2)
        key = jnp.where(iota == riota, maxi, key)
        q = N // 4
        a0, a1 = key[:, 0 * q:1 * q], key[:, 1 * q:2 * q]
        a2, a3 = key[:, 2 * q:3 * q], key[:, 3 * q:4 * q]
        a0, a1 = jnp.minimum(a0, a1), jnp.maximum(a0, a1)
        a2, a3 = jnp.minimum(a2, a3), jnp.maximum(a2, a3)
        a0, a2 = jnp.minimum(a0, a2), jnp.maximum(a0, a2)
        a1, a3 = jnp.minimum(a1, a3), jnp.maximum(a1, a3)
        a1, a2 = jnp.minimum(a1, a2), jnp.maximum(a1, a2)
        cols = [riota]
        for _ in range(K - 1):
            m = jnp.min(a0, axis=1, keepdims=True)
            cols.append(m & 0x7FF)
            mask = a0 == m
            a0 = jnp.where(mask, a1, a0)
            a1 = jnp.where(mask, a2, a1)
            a2 = jnp.where(mask, a3, a2)
            a3 = jnp.where(mask, maxi, a3)
        idx_ref[...] = jnp.concatenate(cols, axis=1) + b * N

    blk0 = row0 // _R2
    return pl.pallas_call(
        body,
        grid=(nrow // _R2,),
        in_specs=[
            pl.BlockSpec((_R2, IN_DIM), lambda i: (blk0 + i, 0)),
            pl.BlockSpec((1, IN_DIM, N), lambda i: (b, 0, 0)),
        ],
        out_specs=pl.BlockSpec((_R2, K), lambda i: (i, 0)),
        out_shape=jax.ShapeDtypeStruct((nrow, K), jnp.int32),
    )(xv2, xvt)


_NW = 32
_CH = 128


def _sc_gather(kv2, idxg):
    nidx = idxg.shape[0] * 128
    per_w = nidx // _NW
    nchunk = per_w // _CH
    mesh = plsc.VectorSubcoreMesh(core_axis_name="c", subcore_axis_name="s")

    @functools.partial(
        pl.kernel,
        mesh=mesh,
        out_type=jax.ShapeDtypeStruct((nidx, E), jnp.int32),
        scratch_types=[
            pltpu.VMEM((nchunk, _CH), jnp.int32),
            pltpu.VMEM((_CH, E), jnp.int32),
            pltpu.VMEM((_CH, E), jnp.int32),
            pltpu.SemaphoreType.DMA,
            pltpu.SemaphoreType.DMA,
            pltpu.SemaphoreType.DMA,
            pltpu.SemaphoreType.DMA,
        ],
    )
    def k(kv_hbm, idx_hbm, gkv_hbm, idx_all, buf0, buf1, g0, g1, s0, s1):
        wid = lax.axis_index("s") * 2 + lax.axis_index("c")
        base = wid * per_w
        pltpu.sync_copy(idx_hbm.at[pl.ds(wid * nchunk, nchunk)], idx_all)
        bufs = (buf0, buf1)
        gsems = (g0, g1)
        ssems = (s0, s1)

        def gath(cn, buf, gsem):
            pltpu.async_copy(kv_hbm.at[idx_all.at[cn]], buf, gsem)

        gath(0, buf0, g0)
        gath(1, buf1, g1)

        def body(i, carry):
            for sl in range(2):
                cn = 2 * i + sl
                buf, gsem, ssem = bufs[sl], gsems[sl], ssems[sl]
                pltpu.make_async_copy(kv_hbm.at[pl.ds(0, _CH)],
                                      buf, gsem).wait()
                cp = pltpu.async_copy(
                    buf, gkv_hbm.at[pl.ds(base + cn * _CH, _CH)], ssem)
                cp.wait()

                @pl.when(cn + 2 < nchunk)
                def _():
                    gath(cn + 2, buf, gsem)

            return carry

        lax.fori_loop(0, nchunk // 2, body, 0)

    return k(kv2, idxg)


_R4 = 256


def _ln_in(xv, g, b):
    m = jnp.mean(xv, axis=1, keepdims=True)
    c = xv - m
    v = jnp.mean(c * c, axis=1, keepdims=True)
    return c * lax.rsqrt(v + 1e-5) * g + b


def _attn_body(q_ref, h0_ref, gkv_ref, ow_ref, ob_ref, ln1g_ref,
               ln1b_ref, fw1_ref, fb1_ref, fw2_ref, fb2_ref, ln2g_ref,
               ln2b_ref, wout_ref, out_ref):
    q = q_ref[...]
    gkv = gkv_ref[...]
    gk_lo, gk_hi = _unpack16_halves(gkv[:, 0:128])
    gv_lo, gv_hi = _unpack16_halves(gkv[:, 128:256])
    seg_lo = (lax.broadcasted_iota(jnp.int32, (128, H), 0) // DH ==
              lax.broadcasted_iota(jnp.int32, (128, H), 1)).astype(jnp.float32)
    seg_hi = ((lax.broadcasted_iota(jnp.int32, (128, H), 0) // DH + 4) ==
              lax.broadcasted_iota(jnp.int32, (128, H), 1)).astype(jnp.float32)
    qb_lo = jnp.broadcast_to(q[:, None, 0:128],
                             (_R4, K, 128)).reshape(_R4 * K, 128)
    qb_hi = jnp.broadcast_to(q[:, None, 128:256],
                             (_R4, K, 128)).reshape(_R4 * K, 128)
    l = (lax.dot_general(gk_lo * qb_lo, seg_lo, (((1,), (0,)), ((), ())),
                         preferred_element_type=jnp.float32) +
         lax.dot_general(gk_hi * qb_hi, seg_hi, (((1,), (0,)), ((), ())),
                         preferred_element_type=jnp.float32)) * (1.0 / math.sqrt(DH))
    l3 = l.reshape(_R4, K, H)
    m = jnp.max(l3, axis=1, keepdims=True)
    e = jnp.exp(l3 - m)
    s = jnp.sum(e, axis=1, keepdims=True)
    w3 = e / s
    w2 = w3.reshape(_R4 * K, H)
    wex_lo = lax.dot_general(w2, seg_lo, (((1,), (1,)), ((), ())),
                             preferred_element_type=jnp.float32)
    wex_hi = lax.dot_general(w2, seg_hi, (((1,), (1,)), ((), ())),
                             preferred_element_type=jnp.float32)
    o_lo = jnp.sum((wex_lo * gv_lo).reshape(_R4, K, 128), axis=1)
    o_hi = jnp.sum((wex_hi * gv_hi).reshape(_R4, K, 128), axis=1)
    o = jnp.concatenate([o_lo, o_hi], axis=1)
    mo = lax.dot_general(o, ow_ref[...], (((1,), (1,)), ((), ())),
                         preferred_element_type=jnp.float32) + ob_ref[...]
    h1 = _ln_in(mo + h0_ref[...], ln1g_ref[...], ln1b_ref[...])
    ffh = jax.nn.gelu(lax.dot_general(h1, fw1_ref[...], (((1,), (1,)), ((), ())),
                                      preferred_element_type=jnp.float32)
                      + fb1_ref[...])
    ffo = lax.dot_general(ffh, fw2_ref[...], (((1,), (1,)), ((), ())),
                          preferred_element_type=jnp.float32) + fb2_ref[...]
    h2 = _ln_in(ffo + h1, ln2g_ref[...], ln2b_ref[...])
    out_ref[...] = lax.dot_general(h2, wout_ref[...], (((1,), (1,)), ((), ())),
                                   preferred_element_type=jnp.float32)


def _attn(q2, h02, gkv, blk0, ow, ob, ln1g, ln1b, fw1, fb1, fw2, fb2, ln2g,
          ln2b, wout):
    nrow = gkv.shape[0] // K
    row = pl.BlockSpec((_R4, E), lambda i: (blk0 + i, 0))
    grow = pl.BlockSpec((_R4 * K, E), lambda i: (i, 0))
    full = lambda s: pl.BlockSpec(s, lambda i: (0,) * len(s))
    return pl.pallas_call(
        _attn_body,
        grid=(nrow // _R4,),
        in_specs=[
            row, row, grow,
            full((E, E)), full((1, E)), full((1, E)), full((1, E)),
            full((FF, E)), full((1, FF)), full((E, FF)), full((1, E)),
            full((1, E)), full((1, E)), full((OUT, E)),
        ],
        out_specs=pl.BlockSpec((_R4, OUT), lambda i: (i, 0)),
        out_shape=jax.ShapeDtypeStruct((nrow, OUT), jnp.float32),
    )(q2, h02, gkv, ow, ob, ln1g, ln1b, fw1, fb1, fw2, fb2, ln2g, ln2b,
      wout)


def kernel(x, x_v, W_emb, in_proj_w, in_proj_b, out_proj_w, out_proj_b,
           ln1_g, ln1_b, ff_w1, ff_b1, ff_w2, ff_b2, ln2_g, ln2_b, W_out):
    x2 = x.reshape(B * N, IN_DIM)
    wq, wk, wv = in_proj_w[0:E], in_proj_w[E:2 * E], in_proj_w[2 * E:3 * E]
    bq = in_proj_b[0:E].reshape(1, E)
    bk = in_proj_b[E:2 * E].reshape(1, E)
    bv = in_proj_b[2 * E:3 * E].reshape(1, E)
    xv2 = x_v.reshape(B * N, IN_DIM)
    h02, q2, kvp, xvt = _prep(x2, xv2, W_emb, wq, wk, wv, bq, bk, bv)
    nslc = 1
    rp = N // nslc
    idxs = []
    for b in range(B):
        for h in range(nslc):
            idxs.append(_knn_batch(xv2, xvt, b, b * N + h * rp, rp))
    gkvs = [_sc_gather(kvp, ix.reshape(-1, 128)) for ix in idxs]
    outs = [
        _attn(q2, h02, gkvs[s], s * (rp // _R4), out_proj_w,
              out_proj_b.reshape(1, E), ln1_g.reshape(1, E),
              ln1_b.reshape(1, E), ff_w1, ff_b1.reshape(1, FF), ff_w2,
              ff_b2.reshape(1, E), ln2_g.reshape(1, E), ln2_b.reshape(1, E),
              W_out)
        for s in range(B * nslc)
    ]
    out2 = jnp.concatenate(outs, axis=0)
    return out2.reshape(B, N, OUT), x_v
